# Initial kernel scaffold; baseline (speedup 1.0000x reference)
import jax
import jax.numpy as jnp
from jax.experimental import pallas as pl


def kernel(player1_id, player2_id, edge_index, emb_table, W1, b1, W2, b2, W3, b3, A1, c1, A2, c2, A3, c3):
    raise NotImplementedError("write your pallas kernel here")


# speedup vs baseline: 4.6923x; 4.6923x over previous
import functools

import jax
import jax.numpy as jnp
from jax import lax
from jax.experimental import pallas as pl
from jax.experimental.pallas import tpu as pltpu
from jax.experimental.pallas import tpu_sc as plsc

D = 128
B = 5000
N = 2 * B
E = 320000
NC, NS = 2, 16
NW = NC * NS
NR = 10240
HALF = NR // NC
ACCR = HALF + 8
RPT = HALF // NS
EPAD = 323584
ECH = EPAD // (NS * 128)

_f32 = jnp.float32
_i32 = jnp.int32

_MESH = plsc.VectorSubcoreMesh(core_axis_name="c", subcore_axis_name="s",
                               num_cores=NC, num_subcores=NS)


def _fill(ref, rows, cols, value):
    lanes = cols // 16
    def body(i, _):
        ref[i // lanes, pl.ds((i % lanes) * 16, 16)] = (
            jnp.full((16,), value, _f32))
        return 0
    lax.fori_loop(0, rows * lanes, body, 0)


def _zero_rpt(src128, acc, s):
    pltpu.sync_copy(src128, acc.at[pl.ds(s * RPT, 128)])
    pltpu.sync_copy(src128, acc.at[pl.ds(s * RPT + 128, 128)])
    pltpu.sync_copy(src128.at[pl.ds(0, 64)], acc.at[pl.ds(s * RPT + 256, 64)])


def _readback(acc, stage128, out_hbm, c, s):
    for off, nrows in ((0, 128), (128, 128), (256, 64)):
        pltpu.sync_copy(acc.at[pl.ds(s * RPT + off, nrows)],
                        stage128.at[pl.ds(0, nrows)])
        pltpu.sync_copy(stage128.at[pl.ds(0, nrows)],
                        out_hbm.at[pl.ds(c * HALF + s * RPT + off, nrows)])


@functools.partial(
    pl.kernel,
    out_type=[jax.ShapeDtypeStruct((NR, D), _f32),
              jax.ShapeDtypeStruct((NC, NS, ECH, 128), _i32)],
    mesh=_MESH,
    scratch_types=[
        pltpu.VMEM((NR // (64 * NW), 64), _i32),
        pltpu.VMEM((64, D), _f32),
        pltpu.VMEM((ECH, 128), _i32),
        pltpu.VMEM((ECH, 128), _i32),
        pltpu.SemaphoreType.DMA,
    ],
)
def _sc_emb(ids_hbm, table_hbm, dst_hbm, emb_hbm, dstt_hbm,
            idx_v, rows_v, didx_v, tidx_v, sem):
    c = lax.axis_index("c")
    s = lax.axis_index("s")
    w = s * NC + c

    pltpu.sync_copy(ids_hbm.at[w], idx_v)

    def gat(j, _):
        pltpu.async_copy(table_hbm.at[idx_v.at[j]], rows_v, sem).wait()
        pltpu.sync_copy(rows_v, emb_hbm.at[pl.ds(w * (NR // NW) + j * 64, 64)])
        return 0
    lax.fori_loop(0, 5, gat, 0)

    pltpu.sync_copy(dst_hbm.at[s], didx_v)
    base = c * HALF

    def rmap(i, _):
        v = didx_v[i // 8, pl.ds((i % 8) * 16, 16)] - base
        ok = (v >= 0) & (v < HALF)
        tidx_v[i // 8, pl.ds((i % 8) * 16, 16)] = jnp.where(ok, v, HALF)
        return 0
    lax.fori_loop(0, ECH * 8, rmap, 0)
    pltpu.sync_copy(tidx_v, dstt_hbm.at[c, s])


@functools.partial(
    pl.kernel,
    out_type=jax.ShapeDtypeStruct((NR, D), _f32),
    mesh=_MESH,
    scratch_types=[
        pltpu.VMEM((ECH, 128), _i32),
        pltpu.VMEM((ECH, 128), _i32),
        pltpu.VMEM((128, D), _f32),
        pltpu.VMEM((128, D), _f32),
        pltpu.VMEM_SHARED((ACCR, D), _f32),
        pltpu.SemaphoreType.DMA,
    ],
)
def _sc_agg(hprime_hbm, src_hbm, dstt_hbm, out_hbm,
            sidx_v, didx_v, rows_v, stage_v, acc, sem):
    c = lax.axis_index("c")
    s = lax.axis_index("s")

    _fill(stage_v, 128, D, 0.0)
    _zero_rpt(stage_v, acc, s)
    plsc.subcore_barrier()

    pltpu.sync_copy(src_hbm.at[s], sidx_v)
    pltpu.sync_copy(dstt_hbm.at[c, s], didx_v)

    def go(j, _):
        pltpu.async_copy(hprime_hbm.at[sidx_v.at[j]], rows_v, sem).wait()
        pltpu.sync_copy(rows_v, acc.at[didx_v.at[j]], add=True)
        return 0
    lax.fori_loop(0, ECH, go, 0)
    plsc.subcore_barrier()

    _readback(acc, stage_v, out_hbm, c, s)


RB = 256


def _dinv_of(deg_blk):
    return lax.rsqrt(deg_blk[:, :1] + 1.0)


def _tc_first_body(x_ref, w_ref, deg_ref, o_ref):
    dinv = _dinv_of(deg_ref[...])
    h = jnp.dot(x_ref[...], w_ref[...], preferred_element_type=_f32)
    o_ref[...] = h * dinv


def _tc_first(x, w, deg):
    return pl.pallas_call(
        _tc_first_body,
        grid=(NR // RB,),
        in_specs=[
            pl.BlockSpec((RB, D), lambda i: (i, 0)),
            pl.BlockSpec((D, D), lambda i: (0, 0)),
            pl.BlockSpec((RB, D), lambda i: (i, 0)),
        ],
        out_specs=pl.BlockSpec((RB, D), lambda i: (i, 0)),
        out_shape=jax.ShapeDtypeStruct((NR, D), _f32),
    )(x, w, deg)


def _tc_mid_body(p_ref, h_ref, deg_ref, b_ref, w_ref, o_ref):
    dinv = _dinv_of(deg_ref[...])
    x = dinv * (p_ref[...] + h_ref[...]) + b_ref[...]
    x = jnp.maximum(x, 0.0)
    o_ref[...] = jnp.dot(x, w_ref[...], preferred_element_type=_f32) * dinv


def _tc_mid(p, hprime, deg, b, w_next):
    return pl.pallas_call(
        _tc_mid_body,
        grid=(NR // RB,),
        in_specs=[
            pl.BlockSpec((RB, D), lambda i: (i, 0)),
            pl.BlockSpec((RB, D), lambda i: (i, 0)),
            pl.BlockSpec((RB, D), lambda i: (i, 0)),
            pl.BlockSpec((1, D), lambda i: (0, 0)),
            pl.BlockSpec((D, D), lambda i: (0, 0)),
        ],
        out_specs=pl.BlockSpec((RB, D), lambda i: (i, 0)),
        out_shape=jax.ShapeDtypeStruct((NR, D), _f32),
    )(p, hprime, deg, b, w_next)


RB2 = 1000


def _tc_head_body(pa, ha, da, pb, hb, db, b3, a1, c1, a2, c2, a3, c3, o_ref):
    dva = _dinv_of(da[...])
    dvb = _dinv_of(db[...])
    xa = dva * (pa[...] + ha[...]) + b3[...]
    xb = dvb * (pb[...] + hb[...]) + b3[...]
    comb = jnp.concatenate([xa, xb], axis=1)
    h = jnp.dot(comb, a1[...], preferred_element_type=_f32) + c1[...]
    h = jnp.maximum(h, 0.0)
    h = jnp.dot(h, a2[...], preferred_element_type=_f32) + c2[...]
    h = jnp.maximum(h, 0.0)
    z = jnp.dot(h, a3[...], preferred_element_type=_f32) + c3[...]
    o_ref[...] = 1.0 / (1.0 + jnp.exp(-z))


def _tc_head(p, hprime, deg, b3, a1, c1, a2, c2, a3, c3):
    nb = B // RB2
    return pl.pallas_call(
        _tc_head_body,
        grid=(B // RB2,),
        in_specs=[
            pl.BlockSpec((RB2, D), lambda i: (i, 0)),
            pl.BlockSpec((RB2, D), lambda i: (i, 0)),
            pl.BlockSpec((RB2, D), lambda i: (i, 0)),
            pl.BlockSpec((RB2, D), lambda i: (i + nb, 0)),
            pl.BlockSpec((RB2, D), lambda i: (i + nb, 0)),
            pl.BlockSpec((RB2, D), lambda i: (i + nb, 0)),
            pl.BlockSpec((1, D), lambda i: (0, 0)),
            pl.BlockSpec((2 * D, 512), lambda i: (0, 0)),
            pl.BlockSpec((1, 512), lambda i: (0, 0)),
            pl.BlockSpec((512, 256), lambda i: (0, 0)),
            pl.BlockSpec((1, 256), lambda i: (0, 0)),
            pl.BlockSpec((256, 1), lambda i: (0, 0)),
            pl.BlockSpec((1, 1), lambda i: (0, 0)),
        ],
        out_specs=pl.BlockSpec((RB2, 1), lambda i: (i, 0)),
        out_shape=jax.ShapeDtypeStruct((B, 1), _f32),
    )(p, hprime, deg, p, hprime, deg, b3, a1, c1, a2, c2, a3, c3)


def kernel(player1_id, player2_id, edge_index, emb_table,
           W1, b1, W2, b2, W3, b3, A1, c1, A2, c2, A3, c3):
    ids = jnp.concatenate([player1_id, player2_id]).astype(_i32)
    ids = jnp.pad(ids, (0, NR - N)).reshape(NW, NR // (64 * NW), 64)
    src = jnp.pad(edge_index[0].astype(_i32), (0, EPAD - E))
    dst = jnp.pad(edge_index[1].astype(_i32), (0, EPAD - E),
                  constant_values=N)
    src = src.reshape(NS, ECH, 128)
    dst = dst.reshape(NS, ECH, 128)

    all_emb, dstt = _sc_emb(ids, emb_table, dst)
    deg = _sc_agg(jnp.ones((NR, D), _f32), src, dstt)

    h1 = _tc_first(all_emb, W1, deg)
    p1 = _sc_agg(h1, src, dstt)
    h2 = _tc_mid(p1, h1, deg, b1.reshape(1, D), W2)
    p2 = _sc_agg(h2, src, dstt)
    h3 = _tc_mid(p2, h2, deg, b2.reshape(1, D), W3)
    p3 = _sc_agg(h3, src, dstt)

    return _tc_head(p3, h3, deg, b3.reshape(1, D),
                    A1, c1.reshape(1, 512), A2, c2.reshape(1, 256),
                    A3, c3.reshape(1, 1))


# speedup vs baseline: 5.6783x; 1.2101x over previous
import functools

import jax
import jax.numpy as jnp
from jax import lax
from jax.experimental import pallas as pl
from jax.experimental.pallas import tpu as pltpu
from jax.experimental.pallas import tpu_sc as plsc

D = 128
B = 5000
N = 2 * B
E = 320000
NC, NS = 2, 16
NW = NC * NS
NR = 10240
HALF = NR // NC
ACCR = HALF + 8
RPT = HALF // NS
EPAD = 323584
ECH = EPAD // (NS * 128)

_f32 = jnp.float32
_i32 = jnp.int32

_MESH = plsc.VectorSubcoreMesh(core_axis_name="c", subcore_axis_name="s",
                               num_cores=NC, num_subcores=NS)


def _fill(ref, rows, cols, value):
    lanes = cols // 16
    def body(i, _):
        ref[i // lanes, pl.ds((i % lanes) * 16, 16)] = (
            jnp.full((16,), value, _f32))
        return 0
    lax.fori_loop(0, rows * lanes, body, 0)


def _zero_rpt(src64, acc, s):
    for k in range(RPT // 64):
        pltpu.sync_copy(src64, acc.at[pl.ds(s * RPT + k * 64, 64)])


def _readback(acc, stage64, out_hbm, c, s):
    for k in range(RPT // 64):
        pltpu.sync_copy(acc.at[pl.ds(s * RPT + k * 64, 64)], stage64)
        pltpu.sync_copy(stage64,
                        out_hbm.at[pl.ds(c * HALF + s * RPT + k * 64, 64)])


@functools.partial(
    pl.kernel,
    out_type=[jax.ShapeDtypeStruct((NR, D), _f32),
              jax.ShapeDtypeStruct((NC, NS, ECH, 128), _i32)],
    mesh=_MESH,
    scratch_types=[
        pltpu.VMEM((NR // (32 * NW), 32), _i32),
        pltpu.VMEM((32, D), _f32),
        pltpu.VMEM((ECH, 128), _i32),
        pltpu.SemaphoreType.DMA,
    ],
)
def _sc_emb(ids_hbm, table_hbm, dst_hbm, emb_hbm, dstt_hbm,
            idx_v, rows_v, didx_v, sem):
    c = lax.axis_index("c")
    s = lax.axis_index("s")
    w = s * NC + c

    pltpu.sync_copy(ids_hbm.at[w], idx_v)

    def gat(j, _):
        pltpu.async_copy(table_hbm.at[idx_v.at[j]], rows_v, sem).wait()
        pltpu.sync_copy(rows_v, emb_hbm.at[pl.ds(w * (NR // NW) + j * 32, 32)])
        return 0
    lax.fori_loop(0, 10, gat, 0)

    pltpu.sync_copy(dst_hbm.at[s], didx_v)
    base = c * HALF

    def rmap(i, _):
        v = didx_v[i // 8, pl.ds((i % 8) * 16, 16)] - base
        ok = (v >= 0) & (v < HALF)
        didx_v[i // 8, pl.ds((i % 8) * 16, 16)] = jnp.where(ok, v, HALF)
        return 0
    lax.fori_loop(0, ECH * 8, rmap, 0)
    pltpu.sync_copy(didx_v, dstt_hbm.at[c, s])


@functools.partial(
    pl.kernel,
    out_type=jax.ShapeDtypeStruct((NR, D), _f32),
    mesh=_MESH,
    scratch_types=[
        pltpu.VMEM((ECH, 128), _i32),
        pltpu.VMEM((128, D), _f32),
        pltpu.VMEM((64, D), _f32),
        pltpu.VMEM_SHARED((ACCR, D), _f32),
        pltpu.SemaphoreType.DMA,
        pltpu.SemaphoreType.DMA,
    ],
)
def _sc_deg(dstt_hbm, out_hbm, didx_v, ones_v, stage_v, acc, semA, semB):
    c = lax.axis_index("c")
    s = lax.axis_index("s")

    _fill(stage_v, 64, D, 0.0)
    _zero_rpt(stage_v, acc, s)
    plsc.subcore_barrier()

    _fill(ones_v, 128, D, 1.0)
    pltpu.sync_copy(dstt_hbm.at[c, s], didx_v)

    def go(g, _):
        dA = pltpu.async_copy(ones_v, acc.at[didx_v.at[2 * g]], semA,
                              add=True)
        dB = pltpu.async_copy(ones_v, acc.at[didx_v.at[2 * g + 1]], semB,
                              add=True)
        dA.wait()
        dB.wait()
        return 0
    lax.fori_loop(0, ECH // 2, go, 0)
    plsc.subcore_barrier()

    _readback(acc, stage_v, out_hbm, c, s)


@functools.partial(
    pl.kernel,
    out_type=jax.ShapeDtypeStruct((NR, D), _f32),
    mesh=_MESH,
    scratch_types=[
        pltpu.VMEM((ECH, 128), _i32),
        pltpu.VMEM((ECH, 128), _i32),
        pltpu.VMEM((128, D), _f32),
        pltpu.VMEM((128, D), _f32),
        pltpu.VMEM((64, D), _f32),
        pltpu.VMEM_SHARED((ACCR, D), _f32),
        pltpu.SemaphoreType.DMA,
        pltpu.SemaphoreType.DMA,
    ],
)
def _sc_agg(hprime_hbm, src_hbm, dstt_hbm, out_hbm,
            sidx_v, didx_v, rows_a, rows_b, stage_v, acc, semA, semB):
    c = lax.axis_index("c")
    s = lax.axis_index("s")

    _fill(stage_v, 64, D, 0.0)
    _zero_rpt(stage_v, acc, s)
    plsc.subcore_barrier()

    pltpu.sync_copy(src_hbm.at[s], sidx_v)
    pltpu.sync_copy(dstt_hbm.at[c, s], didx_v)

    def go(g, _):
        dA = pltpu.async_copy(hprime_hbm.at[sidx_v.at[2 * g]], rows_a, semA)
        dB = pltpu.async_copy(hprime_hbm.at[sidx_v.at[2 * g + 1]], rows_b,
                              semB)
        dA.wait()
        pltpu.sync_copy(rows_a, acc.at[didx_v.at[2 * g]], add=True)
        dB.wait()
        pltpu.sync_copy(rows_b, acc.at[didx_v.at[2 * g + 1]], add=True)
        return 0
    lax.fori_loop(0, ECH // 2, go, 0)
    plsc.subcore_barrier()

    _readback(acc, stage_v, out_hbm, c, s)


RB = 256


def _dinv_of(deg_blk):
    return lax.rsqrt(deg_blk[:, :1] + 1.0)


def _tc_first_body(x_ref, w_ref, deg_ref, o_ref):
    dinv = _dinv_of(deg_ref[...])
    h = jnp.dot(x_ref[...], w_ref[...], preferred_element_type=_f32)
    o_ref[...] = h * dinv


def _tc_first(x, w, deg):
    return pl.pallas_call(
        _tc_first_body,
        grid=(NR // RB,),
        in_specs=[
            pl.BlockSpec((RB, D), lambda i: (i, 0)),
            pl.BlockSpec((D, D), lambda i: (0, 0)),
            pl.BlockSpec((RB, D), lambda i: (i, 0)),
        ],
        out_specs=pl.BlockSpec((RB, D), lambda i: (i, 0)),
        out_shape=jax.ShapeDtypeStruct((NR, D), _f32),
    )(x, w, deg)


def _tc_mid_body(p_ref, h_ref, deg_ref, b_ref, w_ref, o_ref):
    dinv = _dinv_of(deg_ref[...])
    x = dinv * (p_ref[...] + h_ref[...]) + b_ref[...]
    x = jnp.maximum(x, 0.0)
    o_ref[...] = jnp.dot(x, w_ref[...], preferred_element_type=_f32) * dinv


def _tc_mid(p, hprime, deg, b, w_next):
    return pl.pallas_call(
        _tc_mid_body,
        grid=(NR // RB,),
        in_specs=[
            pl.BlockSpec((RB, D), lambda i: (i, 0)),
            pl.BlockSpec((RB, D), lambda i: (i, 0)),
            pl.BlockSpec((RB, D), lambda i: (i, 0)),
            pl.BlockSpec((1, D), lambda i: (0, 0)),
            pl.BlockSpec((D, D), lambda i: (0, 0)),
        ],
        out_specs=pl.BlockSpec((RB, D), lambda i: (i, 0)),
        out_shape=jax.ShapeDtypeStruct((NR, D), _f32),
    )(p, hprime, deg, b, w_next)


RB2 = 1000


def _tc_head_body(pa, ha, da, pb, hb, db, b3, a1, c1, a2, c2, a3, c3, o_ref):
    dva = _dinv_of(da[...])
    dvb = _dinv_of(db[...])
    xa = dva * (pa[...] + ha[...]) + b3[...]
    xb = dvb * (pb[...] + hb[...]) + b3[...]
    comb = jnp.concatenate([xa, xb], axis=1)
    h = jnp.dot(comb, a1[...], preferred_element_type=_f32) + c1[...]
    h = jnp.maximum(h, 0.0)
    h = jnp.dot(h, a2[...], preferred_element_type=_f32) + c2[...]
    h = jnp.maximum(h, 0.0)
    z = jnp.dot(h, a3[...], preferred_element_type=_f32) + c3[...]
    o_ref[...] = 1.0 / (1.0 + jnp.exp(-z))


def _tc_head(p, hprime, deg, b3, a1, c1, a2, c2, a3, c3):
    nb = B // RB2
    return pl.pallas_call(
        _tc_head_body,
        grid=(B // RB2,),
        in_specs=[
            pl.BlockSpec((RB2, D), lambda i: (i, 0)),
            pl.BlockSpec((RB2, D), lambda i: (i, 0)),
            pl.BlockSpec((RB2, D), lambda i: (i, 0)),
            pl.BlockSpec((RB2, D), lambda i: (i + nb, 0)),
            pl.BlockSpec((RB2, D), lambda i: (i + nb, 0)),
            pl.BlockSpec((RB2, D), lambda i: (i + nb, 0)),
            pl.BlockSpec((1, D), lambda i: (0, 0)),
            pl.BlockSpec((2 * D, 512), lambda i: (0, 0)),
            pl.BlockSpec((1, 512), lambda i: (0, 0)),
            pl.BlockSpec((512, 256), lambda i: (0, 0)),
            pl.BlockSpec((1, 256), lambda i: (0, 0)),
            pl.BlockSpec((256, 1), lambda i: (0, 0)),
            pl.BlockSpec((1, 1), lambda i: (0, 0)),
        ],
        out_specs=pl.BlockSpec((RB2, 1), lambda i: (i, 0)),
        out_shape=jax.ShapeDtypeStruct((B, 1), _f32),
    )(p, hprime, deg, p, hprime, deg, b3, a1, c1, a2, c2, a3, c3)


def kernel(player1_id, player2_id, edge_index, emb_table,
           W1, b1, W2, b2, W3, b3, A1, c1, A2, c2, A3, c3):
    ids = jnp.concatenate([player1_id, player2_id]).astype(_i32)
    ids = jnp.pad(ids, (0, NR - N)).reshape(NW, NR // (32 * NW), 32)
    src = jnp.pad(edge_index[0].astype(_i32), (0, EPAD - E))
    dst = jnp.pad(edge_index[1].astype(_i32), (0, EPAD - E),
                  constant_values=N)
    src = src.reshape(NS, ECH, 128)
    dst = dst.reshape(NS, ECH, 128)

    all_emb, dstt = _sc_emb(ids, emb_table, dst)
    deg = _sc_deg(dstt)

    h1 = _tc_first(all_emb, W1, deg)
    p1 = _sc_agg(h1, src, dstt)
    h2 = _tc_mid(p1, h1, deg, b1.reshape(1, D), W2)
    p2 = _sc_agg(h2, src, dstt)
    h3 = _tc_mid(p2, h2, deg, b2.reshape(1, D), W3)
    p3 = _sc_agg(h3, src, dstt)

    return _tc_head(p3, h3, deg, b3.reshape(1, D),
                    A1, c1.reshape(1, 512), A2, c2.reshape(1, 256),
                    A3, c3.reshape(1, 1))


# speedup vs baseline: 5.7606x; 1.0145x over previous
import functools

import jax
import jax.numpy as jnp
from jax import lax
from jax.experimental import pallas as pl
from jax.experimental.pallas import tpu as pltpu
from jax.experimental.pallas import tpu_sc as plsc

D = 128
B = 5000
N = 2 * B
E = 320000
NC, NS = 2, 16
NW = NC * NS
NR = 10240
HALF = NR // NC
ACCR = HALF + 8
RPT = HALF // NS
EPAD = 323584
ECH = EPAD // (NS * 128)

_f32 = jnp.float32
_i32 = jnp.int32

_MESH = plsc.VectorSubcoreMesh(core_axis_name="c", subcore_axis_name="s",
                               num_cores=NC, num_subcores=NS)


def _fill(ref, rows, cols, value):
    lanes = cols // 16
    def body(i, _):
        ref[i // lanes, pl.ds((i % lanes) * 16, 16)] = (
            jnp.full((16,), value, _f32))
        return 0
    lax.fori_loop(0, rows * lanes, body, 0)


def _zero_rpt(src64, acc, s):
    for k in range(RPT // 64):
        pltpu.sync_copy(src64, acc.at[pl.ds(s * RPT + k * 64, 64)])


def _readback(acc, stage64, out_hbm, c, s):
    for k in range(RPT // 64):
        pltpu.sync_copy(acc.at[pl.ds(s * RPT + k * 64, 64)], stage64)
        pltpu.sync_copy(stage64,
                        out_hbm.at[pl.ds(c * HALF + s * RPT + k * 64, 64)])


@functools.partial(
    pl.kernel,
    out_type=[jax.ShapeDtypeStruct((NR, D), _f32),
              jax.ShapeDtypeStruct((NC, NS, ECH, 128), _i32)],
    mesh=_MESH,
    scratch_types=[
        pltpu.VMEM((NR // (32 * NW), 32), _i32),
        pltpu.VMEM((32, D), _f32),
        pltpu.VMEM((ECH, 128), _i32),
        pltpu.SemaphoreType.DMA,
    ],
)
def _sc_emb(ids_hbm, table_hbm, dst_hbm, emb_hbm, dstt_hbm,
            idx_v, rows_v, didx_v, sem):
    c = lax.axis_index("c")
    s = lax.axis_index("s")
    w = s * NC + c

    pltpu.sync_copy(ids_hbm.at[w], idx_v)

    def gat(j, _):
        pltpu.async_copy(table_hbm.at[idx_v.at[j]], rows_v, sem).wait()
        pltpu.sync_copy(rows_v, emb_hbm.at[pl.ds(w * (NR // NW) + j * 32, 32)])
        return 0
    lax.fori_loop(0, 10, gat, 0)

    pltpu.sync_copy(dst_hbm.at[s], didx_v)
    base = c * HALF

    def rmap(i, _):
        v = didx_v[i // 8, pl.ds((i % 8) * 16, 16)] - base
        ok = (v >= 0) & (v < HALF)
        didx_v[i // 8, pl.ds((i % 8) * 16, 16)] = jnp.where(ok, v, HALF)
        return 0
    lax.fori_loop(0, ECH * 8, rmap, 0)
    pltpu.sync_copy(didx_v, dstt_hbm.at[c, s])


@functools.partial(
    pl.kernel,
    out_type=jax.ShapeDtypeStruct((NR, D), _f32),
    mesh=_MESH,
    scratch_types=[
        pltpu.VMEM((ECH, 128), _i32),
        pltpu.VMEM((128, D), _f32),
        pltpu.VMEM((64, D), _f32),
        pltpu.VMEM_SHARED((ACCR, D), _f32),
        pltpu.SemaphoreType.DMA,
        pltpu.SemaphoreType.DMA,
    ],
)
def _sc_deg(dstt_hbm, out_hbm, didx_v, ones_v, stage_v, acc, semA, semB):
    c = lax.axis_index("c")
    s = lax.axis_index("s")

    _fill(stage_v, 64, D, 0.0)
    _zero_rpt(stage_v, acc, s)
    plsc.subcore_barrier()

    _fill(ones_v, 128, D, 1.0)
    pltpu.sync_copy(dstt_hbm.at[c, s], didx_v)

    def go(g, _):
        dA = pltpu.async_copy(ones_v, acc.at[didx_v.at[2 * g]], semA,
                              add=True)
        dB = pltpu.async_copy(ones_v, acc.at[didx_v.at[2 * g + 1]], semB,
                              add=True)
        dA.wait()
        dB.wait()
        return 0
    lax.fori_loop(0, ECH // 2, go, 0)
    plsc.subcore_barrier()

    _readback(acc, stage_v, out_hbm, c, s)


@functools.partial(
    pl.kernel,
    out_type=jax.ShapeDtypeStruct((NR, D), _f32),
    mesh=_MESH,
    scratch_types=[
        pltpu.VMEM((ECH, 128), _i32),
        pltpu.VMEM((ECH, 128), _i32),
        pltpu.VMEM((64, D), _f32),
        pltpu.VMEM((64, D), _f32),
        pltpu.VMEM((64, D), _f32),
        pltpu.VMEM((64, D), _f32),
        pltpu.VMEM((64, D), _f32),
        pltpu.VMEM_SHARED((ACCR, D), _f32),
        pltpu.SemaphoreType.DMA,
        pltpu.SemaphoreType.DMA,
        pltpu.SemaphoreType.DMA,
        pltpu.SemaphoreType.DMA,
        pltpu.SemaphoreType.DMA,
        pltpu.SemaphoreType.DMA,
        pltpu.SemaphoreType.DMA,
        pltpu.SemaphoreType.DMA,
    ],
)
def _sc_agg(hprime_hbm, src_hbm, dstt_hbm, out_hbm,
            sidx_v, didx_v, buf0, buf1, buf2, buf3, stage_v, acc,
            g0, g1, g2, g3, s0, s1, s2, s3):
    c = lax.axis_index("c")
    s = lax.axis_index("s")
    bufs = (buf0, buf1, buf2, buf3)
    gsems = (g0, g1, g2, g3)
    ssems = (s0, s1, s2, s3)

    _fill(stage_v, 64, D, 0.0)
    _zero_rpt(stage_v, acc, s)
    plsc.subcore_barrier()

    pltpu.sync_copy(src_hbm.at[s], sidx_v)
    pltpu.sync_copy(dstt_hbm.at[c, s], didx_v)

    def go(g, _):
        gd = []
        for b in range(4):
            row = 2 * g + (b // 2)
            off = (b % 2) * 64
            gd.append(pltpu.async_copy(
                hprime_hbm.at[sidx_v.at[row, pl.ds(off, 64)]],
                bufs[b], gsems[b]))
        sd = []
        for b in range(4):
            row = 2 * g + (b // 2)
            off = (b % 2) * 64
            gd[b].wait()
            sd.append(pltpu.async_copy(
                bufs[b], acc.at[didx_v.at[row, pl.ds(off, 64)]],
                ssems[b], add=True))
        for b in range(4):
            sd[b].wait()
        return 0
    lax.fori_loop(0, ECH // 2, go, 0)
    plsc.subcore_barrier()

    _readback(acc, stage_v, out_hbm, c, s)


RB = 256


def _dinv_of(deg_blk):
    return lax.rsqrt(deg_blk[:, :1] + 1.0)


def _tc_first_body(x_ref, w_ref, deg_ref, o_ref):
    dinv = _dinv_of(deg_ref[...])
    h = jnp.dot(x_ref[...], w_ref[...], preferred_element_type=_f32)
    o_ref[...] = h * dinv


def _tc_first(x, w, deg):
    return pl.pallas_call(
        _tc_first_body,
        grid=(NR // RB,),
        in_specs=[
            pl.BlockSpec((RB, D), lambda i: (i, 0)),
            pl.BlockSpec((D, D), lambda i: (0, 0)),
            pl.BlockSpec((RB, D), lambda i: (i, 0)),
        ],
        out_specs=pl.BlockSpec((RB, D), lambda i: (i, 0)),
        out_shape=jax.ShapeDtypeStruct((NR, D), _f32),
    )(x, w, deg)


def _tc_mid_body(p_ref, h_ref, deg_ref, b_ref, w_ref, o_ref):
    dinv = _dinv_of(deg_ref[...])
    x = dinv * (p_ref[...] + h_ref[...]) + b_ref[...]
    x = jnp.maximum(x, 0.0)
    o_ref[...] = jnp.dot(x, w_ref[...], preferred_element_type=_f32) * dinv


def _tc_mid(p, hprime, deg, b, w_next):
    return pl.pallas_call(
        _tc_mid_body,
        grid=(NR // RB,),
        in_specs=[
            pl.BlockSpec((RB, D), lambda i: (i, 0)),
            pl.BlockSpec((RB, D), lambda i: (i, 0)),
            pl.BlockSpec((RB, D), lambda i: (i, 0)),
            pl.BlockSpec((1, D), lambda i: (0, 0)),
            pl.BlockSpec((D, D), lambda i: (0, 0)),
        ],
        out_specs=pl.BlockSpec((RB, D), lambda i: (i, 0)),
        out_shape=jax.ShapeDtypeStruct((NR, D), _f32),
    )(p, hprime, deg, b, w_next)


RB2 = 1000


def _tc_head_body(pa, ha, da, pb, hb, db, b3, a1, c1, a2, c2, a3, c3, o_ref):
    dva = _dinv_of(da[...])
    dvb = _dinv_of(db[...])
    xa = dva * (pa[...] + ha[...]) + b3[...]
    xb = dvb * (pb[...] + hb[...]) + b3[...]
    comb = jnp.concatenate([xa, xb], axis=1)
    h = jnp.dot(comb, a1[...], preferred_element_type=_f32) + c1[...]
    h = jnp.maximum(h, 0.0)
    h = jnp.dot(h, a2[...], preferred_element_type=_f32) + c2[...]
    h = jnp.maximum(h, 0.0)
    z = jnp.dot(h, a3[...], preferred_element_type=_f32) + c3[...]
    o_ref[...] = 1.0 / (1.0 + jnp.exp(-z))


def _tc_head(p, hprime, deg, b3, a1, c1, a2, c2, a3, c3):
    nb = B // RB2
    return pl.pallas_call(
        _tc_head_body,
        grid=(B // RB2,),
        in_specs=[
            pl.BlockSpec((RB2, D), lambda i: (i, 0)),
            pl.BlockSpec((RB2, D), lambda i: (i, 0)),
            pl.BlockSpec((RB2, D), lambda i: (i, 0)),
            pl.BlockSpec((RB2, D), lambda i: (i + nb, 0)),
            pl.BlockSpec((RB2, D), lambda i: (i + nb, 0)),
            pl.BlockSpec((RB2, D), lambda i: (i + nb, 0)),
            pl.BlockSpec((1, D), lambda i: (0, 0)),
            pl.BlockSpec((2 * D, 512), lambda i: (0, 0)),
            pl.BlockSpec((1, 512), lambda i: (0, 0)),
            pl.BlockSpec((512, 256), lambda i: (0, 0)),
            pl.BlockSpec((1, 256), lambda i: (0, 0)),
            pl.BlockSpec((256, 1), lambda i: (0, 0)),
            pl.BlockSpec((1, 1), lambda i: (0, 0)),
        ],
        out_specs=pl.BlockSpec((RB2, 1), lambda i: (i, 0)),
        out_shape=jax.ShapeDtypeStruct((B, 1), _f32),
    )(p, hprime, deg, p, hprime, deg, b3, a1, c1, a2, c2, a3, c3)


def kernel(player1_id, player2_id, edge_index, emb_table,
           W1, b1, W2, b2, W3, b3, A1, c1, A2, c2, A3, c3):
    ids = jnp.concatenate([player1_id, player2_id]).astype(_i32)
    ids = jnp.pad(ids, (0, NR - N)).reshape(NW, NR // (32 * NW), 32)
    src = jnp.pad(edge_index[0].astype(_i32), (0, EPAD - E))
    dst = jnp.pad(edge_index[1].astype(_i32), (0, EPAD - E),
                  constant_values=N)
    src = src.reshape(NS, ECH, 128)
    dst = dst.reshape(NS, ECH, 128)

    all_emb, dstt = _sc_emb(ids, emb_table, dst)
    deg = _sc_deg(dstt)

    h1 = _tc_first(all_emb, W1, deg)
    p1 = _sc_agg(h1, src, dstt)
    h2 = _tc_mid(p1, h1, deg, b1.reshape(1, D), W2)
    p2 = _sc_agg(h2, src, dstt)
    h3 = _tc_mid(p2, h2, deg, b2.reshape(1, D), W3)
    p3 = _sc_agg(h3, src, dstt)

    return _tc_head(p3, h3, deg, b3.reshape(1, D),
                    A1, c1.reshape(1, 512), A2, c2.reshape(1, 256),
                    A3, c3.reshape(1, 1))
